# baseline (speedup 1.0000x reference)
import functools

import jax
import jax.numpy as jnp
from jax import lax
from jax.experimental import pallas as pl
from jax.experimental.pallas import tpu as pltpu
from jax.experimental.pallas import tpu_sc as plsc

NCLS = 7
B_TOTAL = 1048576
UNC_THR = 0.5
DEC_THR = 0.25
SPEC_W = 0.7

_R_SC = 393216
_R_TC = B_TOTAL - _R_SC

_NC = 2
_NS = 16
_NW = _NC * _NS
_ROWS_PER_W = _R_SC // _NW
_CH = 4096
_N_CHUNKS = _ROWS_PER_W // _CH
_VECS = _CH // 16


def _sc_body(probs_hbm, cu_hbm, sr_hbm, pr_hbm, out_hbm,
             pv0, pv1, cv0, cv1, sv0, sv1, rv0, rv1, ov0, ov1,
             isem, osem):
    wid = lax.axis_index("s") * _NC + lax.axis_index("c")
    w0 = wid * _ROWS_PER_W

    pv = (pv0, pv1)
    cv = (cv0, cv1)
    sv = (sv0, sv1)
    rv = (rv0, rv1)
    ov = (ov0, ov1)

    def in_copies(t, s):
        base = w0 + t * _CH
        sl = pl.ds(base, _CH)
        return (
            pltpu.make_async_copy(probs_hbm.at[pl.ds(0, NCLS), sl], pv[s], isem.at[s, 0]),
            pltpu.make_async_copy(cu_hbm.at[sl], cv[s], isem.at[s, 1]),
            pltpu.make_async_copy(sr_hbm.at[sl], sv[s], isem.at[s, 2]),
            pltpu.make_async_copy(pr_hbm.at[sl], rv[s], isem.at[s, 3]),
        )

    def out_copy(t, s):
        base = w0 + t * _CH
        return pltpu.make_async_copy(ov[s], out_hbm.at[pl.ds(base, _CH)], osem.at[s])

    def compute(s):
        pvs, cvs, svs, rvs, ovs = pv[s], cv[s], sv[s], rv[s], ov[s]

        def vec_body(j, _):
            r0 = j * 16
            sl = pl.ds(r0, 16)
            mv = pvs[0, sl]
            mi = jnp.zeros((16,), jnp.int32)
            for c in range(1, NCLS):
                g = pvs[c, sl]
                p = g > mv
                mv = jnp.where(p, g, mv)
                mi = jnp.where(p, c, mi)

            cu = cvs[sl]
            sr = svs[sl]
            pr = rvs[sl]

            dm = jnp.abs(sr - pr) > DEC_THR
            us = sr > pr
            spec = jnp.maximum(1.0 - sr, SPEC_W * (1.0 - pr))
            spat = jnp.maximum(1.0 - pr, SPEC_W * (1.0 - sr))
            fu = jnp.where(dm, jnp.where(us, spec, spat), cu)
            rs = SPEC_W * fu + (1.0 - mv)
            unk = rs > UNC_THR
            ovs[sl] = jnp.where(unk, NCLS, mi)
            return 0

        lax.fori_loop(0, _VECS, vec_body, 0)

    for c in in_copies(0, 0):
        c.start()
    for t in range(_N_CHUNKS):
        s = t % 2
        if t + 1 < _N_CHUNKS:
            for c in in_copies(t + 1, (t + 1) % 2):
                c.start()
        for c in in_copies(t, s):
            c.wait()
        if t >= 2:
            out_copy(t - 2, s).wait()
        compute(s)
        out_copy(t, s).start()
    out_copy(_N_CHUNKS - 2, (_N_CHUNKS - 2) % 2).wait()
    out_copy(_N_CHUNKS - 1, (_N_CHUNKS - 1) % 2).wait()


_sc_call = functools.partial(
    pl.kernel,
    out_type=jax.ShapeDtypeStruct((_R_SC,), jnp.int32),
    mesh=plsc.VectorSubcoreMesh(core_axis_name="c", subcore_axis_name="s"),
    compiler_params=pltpu.CompilerParams(
        needs_layout_passes=False,
        use_tc_tiling_on_sc=True,
    ),
    scratch_types=(
        [pltpu.VMEM((NCLS, _CH), jnp.float32)] * 2
        + [pltpu.VMEM((_CH,), jnp.float32)] * 6
        + [pltpu.VMEM((_CH,), jnp.int32)] * 2
        + [pltpu.SemaphoreType.DMA((2, 4)), pltpu.SemaphoreType.DMA((2,))]
    ),
)(_sc_body)


_BR = 64
_TBLK = _BR * 128


def _tc_body(p_ref, cu_ref, sr_ref, pr_ref, o_ref):
    x = p_ref[...]
    mx = jnp.max(x, axis=0, keepdims=True)
    iota7 = jax.lax.broadcasted_iota(jnp.int32, (NCLS, _TBLK), 0)
    mi = jnp.min(jnp.where(x == mx, iota7, NCLS), axis=0, keepdims=True)
    mxr = mx.reshape(_BR, 128)
    mir = mi.reshape(_BR, 128)

    cu = cu_ref[...]
    sr = sr_ref[...]
    pr = pr_ref[...]
    dm = jnp.abs(sr - pr) > DEC_THR
    us = sr > pr
    spec = jnp.maximum(1.0 - sr, SPEC_W * (1.0 - pr))
    spat = jnp.maximum(1.0 - pr, SPEC_W * (1.0 - sr))
    fu = jnp.where(dm, jnp.where(us, spec, spat), cu)
    rs = SPEC_W * fu + (1.0 - mxr)
    unk = rs > UNC_THR
    o_ref[...] = jnp.where(unk, NCLS, mir)


_OFF_B = _R_SC // _TBLK
_OFF_R = _R_SC // 128 // _BR

_tc_call = pl.pallas_call(
    _tc_body,
    grid=(_R_TC // _TBLK,),
    in_specs=[
        pl.BlockSpec((NCLS, _TBLK), lambda i: (0, _OFF_B + i)),
        pl.BlockSpec((_BR, 128), lambda i: (_OFF_R + i, 0)),
        pl.BlockSpec((_BR, 128), lambda i: (_OFF_R + i, 0)),
        pl.BlockSpec((_BR, 128), lambda i: (_OFF_R + i, 0)),
    ],
    out_specs=pl.BlockSpec((_BR, 128), lambda i: (i, 0)),
    out_shape=jax.ShapeDtypeStruct((_R_TC // 128, 128), jnp.int32),
)


def kernel(probs, uncertainty_combined, spectral_reliability, spatial_reliability):
    pt = probs.T
    cu = uncertainty_combined.reshape(-1)
    sr = spectral_reliability.reshape(-1)
    pr = spatial_reliability.reshape(-1)
    cu2 = cu.reshape(-1, 128)
    sr2 = sr.reshape(-1, 128)
    pr2 = pr.reshape(-1, 128)
    sc_out = _sc_call(pt, cu, sr, pr)
    tc_out = _tc_call(pt, cu2, sr2, pr2)
    return jnp.concatenate([sc_out, tc_out.reshape(-1)])


# speedup vs baseline: 1.2487x; 1.2487x over previous
import functools

import jax
import jax.numpy as jnp
from jax import lax
from jax.experimental import pallas as pl
from jax.experimental.pallas import tpu as pltpu
from jax.experimental.pallas import tpu_sc as plsc

NCLS = 7
B_TOTAL = 1048576
UNC_THR = 0.5
DEC_THR = 0.25
SPEC_W = 0.7

_R_SC = 589824
_R_TC = B_TOTAL - _R_SC

_NC = 2
_NS = 16
_NW = _NC * _NS
_ROWS_PER_W = _R_SC // _NW
_CH = 2048
_N_CHUNKS = _ROWS_PER_W // _CH
_VECS = _CH // 16


def _sc_body(probs_hbm, cu_hbm, sr_hbm, pr_hbm, out_hbm,
             pv0, pv1, cv0, cv1, sv0, sv1, rv0, rv1, ov0, ov1,
             isem, osem):
    wid = lax.axis_index("s") * _NC + lax.axis_index("c")
    w0 = wid * _ROWS_PER_W

    pv = (pv0, pv1)
    cv = (cv0, cv1)
    sv = (sv0, sv1)
    rv = (rv0, rv1)
    ov = (ov0, ov1)

    def in_copies(t, s):
        base = w0 + t * _CH
        sl = pl.ds(base, _CH)
        return (
            pltpu.make_async_copy(probs_hbm.at[pl.ds(0, NCLS), sl], pv[s], isem.at[s, 0]),
            pltpu.make_async_copy(cu_hbm.at[sl], cv[s], isem.at[s, 1]),
            pltpu.make_async_copy(sr_hbm.at[sl], sv[s], isem.at[s, 2]),
            pltpu.make_async_copy(pr_hbm.at[sl], rv[s], isem.at[s, 3]),
        )

    def out_copy(t, s):
        base = w0 + t * _CH
        return pltpu.make_async_copy(ov[s], out_hbm.at[pl.ds(base, _CH)], osem.at[s])

    def compute(s):
        pvs, cvs, svs, rvs, ovs = pv[s], cv[s], sv[s], rv[s], ov[s]

        def vec_body(j, _):
            r0 = j * 16
            sl = pl.ds(r0, 16)
            mv = pvs[0, sl]
            mi = jnp.zeros((16,), jnp.int32)
            for c in range(1, NCLS):
                g = pvs[c, sl]
                p = g > mv
                mv = jnp.where(p, g, mv)
                mi = jnp.where(p, c, mi)

            cu = cvs[sl]
            sr = svs[sl]
            pr = rvs[sl]

            dm = jnp.abs(sr - pr) > DEC_THR
            us = sr > pr
            spec = jnp.maximum(1.0 - sr, SPEC_W * (1.0 - pr))
            spat = jnp.maximum(1.0 - pr, SPEC_W * (1.0 - sr))
            fu = jnp.where(dm, jnp.where(us, spec, spat), cu)
            rs = SPEC_W * fu + (1.0 - mv)
            unk = rs > UNC_THR
            ovs[sl] = jnp.where(unk, NCLS, mi)
            return 0

        lax.fori_loop(0, _VECS, vec_body, 0)

    for c in in_copies(0, 0):
        c.start()
    for t in range(_N_CHUNKS):
        s = t % 2
        if t + 1 < _N_CHUNKS:
            for c in in_copies(t + 1, (t + 1) % 2):
                c.start()
        for c in in_copies(t, s):
            c.wait()
        if t >= 2:
            out_copy(t - 2, s).wait()
        compute(s)
        out_copy(t, s).start()
    out_copy(_N_CHUNKS - 2, (_N_CHUNKS - 2) % 2).wait()
    out_copy(_N_CHUNKS - 1, (_N_CHUNKS - 1) % 2).wait()


_sc_call = functools.partial(
    pl.kernel,
    out_type=jax.ShapeDtypeStruct((_R_SC,), jnp.int32),
    mesh=plsc.VectorSubcoreMesh(core_axis_name="c", subcore_axis_name="s"),
    compiler_params=pltpu.CompilerParams(
        needs_layout_passes=False,
        use_tc_tiling_on_sc=True,
    ),
    scratch_types=(
        [pltpu.VMEM((NCLS, _CH), jnp.float32)] * 2
        + [pltpu.VMEM((_CH,), jnp.float32)] * 6
        + [pltpu.VMEM((_CH,), jnp.int32)] * 2
        + [pltpu.SemaphoreType.DMA((2, 4)), pltpu.SemaphoreType.DMA((2,))]
    ),
)(_sc_body)


_BR = 64
_TBLK = _BR * 128


def _tc_body(p_ref, cu_ref, sr_ref, pr_ref, o_ref):
    x = p_ref[...]
    mx = jnp.max(x, axis=0, keepdims=True)
    iota7 = jax.lax.broadcasted_iota(jnp.int32, (NCLS, _TBLK), 0)
    mi = jnp.min(jnp.where(x == mx, iota7, NCLS), axis=0, keepdims=True)
    mxr = mx.reshape(_BR, 128)
    mir = mi.reshape(_BR, 128)

    cu = cu_ref[...]
    sr = sr_ref[...]
    pr = pr_ref[...]
    dm = jnp.abs(sr - pr) > DEC_THR
    us = sr > pr
    spec = jnp.maximum(1.0 - sr, SPEC_W * (1.0 - pr))
    spat = jnp.maximum(1.0 - pr, SPEC_W * (1.0 - sr))
    fu = jnp.where(dm, jnp.where(us, spec, spat), cu)
    rs = SPEC_W * fu + (1.0 - mxr)
    unk = rs > UNC_THR
    o_ref[...] = jnp.where(unk, NCLS, mir)


_OFF_B = _R_SC // _TBLK
_OFF_R = _R_SC // 128 // _BR

_tc_call = pl.pallas_call(
    _tc_body,
    grid=(_R_TC // _TBLK,),
    in_specs=[
        pl.BlockSpec((NCLS, _TBLK), lambda i: (0, _OFF_B + i)),
        pl.BlockSpec((_BR, 128), lambda i: (_OFF_R + i, 0)),
        pl.BlockSpec((_BR, 128), lambda i: (_OFF_R + i, 0)),
        pl.BlockSpec((_BR, 128), lambda i: (_OFF_R + i, 0)),
    ],
    out_specs=pl.BlockSpec((_BR, 128), lambda i: (i, 0)),
    out_shape=jax.ShapeDtypeStruct((_R_TC // 128, 128), jnp.int32),
)


def kernel(probs, uncertainty_combined, spectral_reliability, spatial_reliability):
    pt = probs.T
    cu = uncertainty_combined.reshape(-1)
    sr = spectral_reliability.reshape(-1)
    pr = spatial_reliability.reshape(-1)
    cu2 = cu.reshape(-1, 128)
    sr2 = sr.reshape(-1, 128)
    pr2 = pr.reshape(-1, 128)
    sc_out = _sc_call(pt, cu, sr, pr)
    tc_out = _tc_call(pt, cu2, sr2, pr2)
    return jnp.concatenate([sc_out, tc_out.reshape(-1)])


# speedup vs baseline: 1.4604x; 1.1695x over previous
import functools

import jax
import jax.numpy as jnp
from jax import lax
from jax.experimental import pallas as pl
from jax.experimental.pallas import tpu as pltpu
from jax.experimental.pallas import tpu_sc as plsc

NCLS = 7
B_TOTAL = 1048576
UNC_THR = 0.5
DEC_THR = 0.25
SPEC_W = 0.7

_R_SC = 786432
_R_TC = B_TOTAL - _R_SC

_NC = 2
_NS = 16
_NW = _NC * _NS
_ROWS_PER_W = _R_SC // _NW
_CH = 2048
_N_CHUNKS = _ROWS_PER_W // _CH
_VECS = _CH // 16


def _sc_body(probs_hbm, cu_hbm, sr_hbm, pr_hbm, out_hbm,
             pv0, pv1, cv0, cv1, sv0, sv1, rv0, rv1, ov0, ov1,
             isem, osem):
    wid = lax.axis_index("s") * _NC + lax.axis_index("c")
    w0 = wid * _ROWS_PER_W

    pv = (pv0, pv1)
    cv = (cv0, cv1)
    sv = (sv0, sv1)
    rv = (rv0, rv1)
    ov = (ov0, ov1)

    def in_copies(t, s):
        base = w0 + t * _CH
        sl = pl.ds(base, _CH)
        return (
            pltpu.make_async_copy(probs_hbm.at[pl.ds(0, NCLS), sl], pv[s], isem.at[s, 0]),
            pltpu.make_async_copy(cu_hbm.at[sl], cv[s], isem.at[s, 1]),
            pltpu.make_async_copy(sr_hbm.at[sl], sv[s], isem.at[s, 2]),
            pltpu.make_async_copy(pr_hbm.at[sl], rv[s], isem.at[s, 3]),
        )

    def out_copy(t, s):
        base = w0 + t * _CH
        return pltpu.make_async_copy(ov[s], out_hbm.at[pl.ds(base, _CH)], osem.at[s])

    def compute(s):
        pvs, cvs, svs, rvs, ovs = pv[s], cv[s], sv[s], rv[s], ov[s]

        def vec_body(j, _):
            r0 = j * 16
            sl = pl.ds(r0, 16)
            mv = pvs[0, sl]
            mi = jnp.zeros((16,), jnp.int32)
            for c in range(1, NCLS):
                g = pvs[c, sl]
                p = g > mv
                mv = jnp.where(p, g, mv)
                mi = jnp.where(p, c, mi)

            cu = cvs[sl]
            sr = svs[sl]
            pr = rvs[sl]

            dm = jnp.abs(sr - pr) > DEC_THR
            us = sr > pr
            spec = jnp.maximum(1.0 - sr, SPEC_W * (1.0 - pr))
            spat = jnp.maximum(1.0 - pr, SPEC_W * (1.0 - sr))
            fu = jnp.where(dm, jnp.where(us, spec, spat), cu)
            rs = SPEC_W * fu + (1.0 - mv)
            unk = rs > UNC_THR
            ovs[sl] = jnp.where(unk, NCLS, mi)
            return 0

        lax.fori_loop(0, _VECS, vec_body, 0)

    for c in in_copies(0, 0):
        c.start()
    for t in range(_N_CHUNKS):
        s = t % 2
        if t + 1 < _N_CHUNKS:
            for c in in_copies(t + 1, (t + 1) % 2):
                c.start()
        for c in in_copies(t, s):
            c.wait()
        if t >= 2:
            out_copy(t - 2, s).wait()
        compute(s)
        out_copy(t, s).start()
    out_copy(_N_CHUNKS - 2, (_N_CHUNKS - 2) % 2).wait()
    out_copy(_N_CHUNKS - 1, (_N_CHUNKS - 1) % 2).wait()


_sc_call = functools.partial(
    pl.kernel,
    out_type=jax.ShapeDtypeStruct((_R_SC,), jnp.int32),
    mesh=plsc.VectorSubcoreMesh(core_axis_name="c", subcore_axis_name="s"),
    compiler_params=pltpu.CompilerParams(
        needs_layout_passes=False,
        use_tc_tiling_on_sc=True,
    ),
    scratch_types=(
        [pltpu.VMEM((NCLS, _CH), jnp.float32)] * 2
        + [pltpu.VMEM((_CH,), jnp.float32)] * 6
        + [pltpu.VMEM((_CH,), jnp.int32)] * 2
        + [pltpu.SemaphoreType.DMA((2, 4)), pltpu.SemaphoreType.DMA((2,))]
    ),
)(_sc_body)


_BR = 64
_TBLK = _BR * 128


def _tc_body(p_ref, cu_ref, sr_ref, pr_ref, o_ref):
    x = p_ref[...]
    mx = jnp.max(x, axis=0, keepdims=True)
    iota7 = jax.lax.broadcasted_iota(jnp.int32, (NCLS, _TBLK), 0)
    mi = jnp.min(jnp.where(x == mx, iota7, NCLS), axis=0, keepdims=True)
    mxr = mx.reshape(_BR, 128)
    mir = mi.reshape(_BR, 128)

    cu = cu_ref[...]
    sr = sr_ref[...]
    pr = pr_ref[...]
    dm = jnp.abs(sr - pr) > DEC_THR
    us = sr > pr
    spec = jnp.maximum(1.0 - sr, SPEC_W * (1.0 - pr))
    spat = jnp.maximum(1.0 - pr, SPEC_W * (1.0 - sr))
    fu = jnp.where(dm, jnp.where(us, spec, spat), cu)
    rs = SPEC_W * fu + (1.0 - mxr)
    unk = rs > UNC_THR
    o_ref[...] = jnp.where(unk, NCLS, mir)


_OFF_B = _R_SC // _TBLK
_OFF_R = _R_SC // 128 // _BR

_tc_call = pl.pallas_call(
    _tc_body,
    grid=(_R_TC // _TBLK,),
    in_specs=[
        pl.BlockSpec((NCLS, _TBLK), lambda i: (0, _OFF_B + i)),
        pl.BlockSpec((_BR, 128), lambda i: (_OFF_R + i, 0)),
        pl.BlockSpec((_BR, 128), lambda i: (_OFF_R + i, 0)),
        pl.BlockSpec((_BR, 128), lambda i: (_OFF_R + i, 0)),
    ],
    out_specs=pl.BlockSpec((_BR, 128), lambda i: (i, 0)),
    out_shape=jax.ShapeDtypeStruct((_R_TC // 128, 128), jnp.int32),
)


def kernel(probs, uncertainty_combined, spectral_reliability, spatial_reliability):
    pt = probs.T
    cu = uncertainty_combined.reshape(-1)
    sr = spectral_reliability.reshape(-1)
    pr = spatial_reliability.reshape(-1)
    cu2 = cu.reshape(-1, 128)
    sr2 = sr.reshape(-1, 128)
    pr2 = pr.reshape(-1, 128)
    sc_out = _sc_call(pt, cu, sr, pr)
    tc_out = _tc_call(pt, cu2, sr2, pr2)
    return jnp.concatenate([sc_out, tc_out.reshape(-1)])


# speedup vs baseline: 1.4796x; 1.0131x over previous
import functools

import jax
import jax.numpy as jnp
from jax import lax
from jax.experimental import pallas as pl
from jax.experimental.pallas import tpu as pltpu
from jax.experimental.pallas import tpu_sc as plsc

NCLS = 7
B_TOTAL = 1048576
UNC_THR = 0.5
DEC_THR = 0.25
SPEC_W = 0.7

_R_SC = 720896
_R_TC = B_TOTAL - _R_SC

_NC = 2
_NS = 16
_NW = _NC * _NS
_ROWS_PER_W = _R_SC // _NW
_CH = 2048
_N_CHUNKS = _ROWS_PER_W // _CH
_VECS = _CH // 16


def _sc_body(probs_hbm, cu_hbm, sr_hbm, pr_hbm, out_hbm,
             pv0, pv1, cv0, cv1, sv0, sv1, rv0, rv1, ov0, ov1,
             isem, osem):
    wid = lax.axis_index("s") * _NC + lax.axis_index("c")
    w0 = wid * _ROWS_PER_W

    pv = (pv0, pv1)
    cv = (cv0, cv1)
    sv = (sv0, sv1)
    rv = (rv0, rv1)
    ov = (ov0, ov1)

    def in_copies(t, s):
        base = w0 + t * _CH
        sl = pl.ds(base, _CH)
        return (
            pltpu.make_async_copy(probs_hbm.at[pl.ds(0, NCLS), sl], pv[s], isem.at[s, 0]),
            pltpu.make_async_copy(cu_hbm.at[sl], cv[s], isem.at[s, 1]),
            pltpu.make_async_copy(sr_hbm.at[sl], sv[s], isem.at[s, 2]),
            pltpu.make_async_copy(pr_hbm.at[sl], rv[s], isem.at[s, 3]),
        )

    def out_copy(t, s):
        base = w0 + t * _CH
        return pltpu.make_async_copy(ov[s], out_hbm.at[pl.ds(base, _CH)], osem.at[s])

    def compute(s):
        pvs, cvs, svs, rvs, ovs = pv[s], cv[s], sv[s], rv[s], ov[s]

        def vec_body(j, _):
            r0 = j * 16
            sl = pl.ds(r0, 16)
            mv = pvs[0, sl]
            mi = jnp.zeros((16,), jnp.int32)
            for c in range(1, NCLS):
                g = pvs[c, sl]
                p = g > mv
                mv = jnp.where(p, g, mv)
                mi = jnp.where(p, c, mi)

            cu = cvs[sl]
            sr = svs[sl]
            pr = rvs[sl]

            dm = jnp.abs(sr - pr) > DEC_THR
            us = sr > pr
            spec = jnp.maximum(1.0 - sr, SPEC_W * (1.0 - pr))
            spat = jnp.maximum(1.0 - pr, SPEC_W * (1.0 - sr))
            fu = jnp.where(dm, jnp.where(us, spec, spat), cu)
            rs = SPEC_W * fu + (1.0 - mv)
            unk = rs > UNC_THR
            ovs[sl] = jnp.where(unk, NCLS, mi)
            return 0

        lax.fori_loop(0, _VECS, vec_body, 0)

    for c in in_copies(0, 0):
        c.start()
    for t in range(_N_CHUNKS):
        s = t % 2
        if t + 1 < _N_CHUNKS:
            for c in in_copies(t + 1, (t + 1) % 2):
                c.start()
        for c in in_copies(t, s):
            c.wait()
        if t >= 2:
            out_copy(t - 2, s).wait()
        compute(s)
        out_copy(t, s).start()
    out_copy(_N_CHUNKS - 2, (_N_CHUNKS - 2) % 2).wait()
    out_copy(_N_CHUNKS - 1, (_N_CHUNKS - 1) % 2).wait()


_sc_call = functools.partial(
    pl.kernel,
    out_type=jax.ShapeDtypeStruct((_R_SC,), jnp.int32),
    mesh=plsc.VectorSubcoreMesh(core_axis_name="c", subcore_axis_name="s"),
    compiler_params=pltpu.CompilerParams(
        needs_layout_passes=False,
        use_tc_tiling_on_sc=True,
    ),
    scratch_types=(
        [pltpu.VMEM((NCLS, _CH), jnp.float32)] * 2
        + [pltpu.VMEM((_CH,), jnp.float32)] * 6
        + [pltpu.VMEM((_CH,), jnp.int32)] * 2
        + [pltpu.SemaphoreType.DMA((2, 4)), pltpu.SemaphoreType.DMA((2,))]
    ),
)(_sc_body)


_BR = 64
_TBLK = _BR * 128


def _tc_body(p_ref, cu_ref, sr_ref, pr_ref, o_ref):
    x = p_ref[...]
    mx = jnp.max(x, axis=0, keepdims=True)
    iota7 = jax.lax.broadcasted_iota(jnp.int32, (NCLS, _TBLK), 0)
    mi = jnp.min(jnp.where(x == mx, iota7, NCLS), axis=0, keepdims=True)
    mxr = mx.reshape(_BR, 128)
    mir = mi.reshape(_BR, 128)

    cu = cu_ref[...]
    sr = sr_ref[...]
    pr = pr_ref[...]
    dm = jnp.abs(sr - pr) > DEC_THR
    us = sr > pr
    spec = jnp.maximum(1.0 - sr, SPEC_W * (1.0 - pr))
    spat = jnp.maximum(1.0 - pr, SPEC_W * (1.0 - sr))
    fu = jnp.where(dm, jnp.where(us, spec, spat), cu)
    rs = SPEC_W * fu + (1.0 - mxr)
    unk = rs > UNC_THR
    o_ref[...] = jnp.where(unk, NCLS, mir)


_OFF_B = _R_SC // _TBLK
_OFF_R = _R_SC // 128 // _BR

_tc_call = pl.pallas_call(
    _tc_body,
    grid=(_R_TC // _TBLK,),
    in_specs=[
        pl.BlockSpec((NCLS, _TBLK), lambda i: (0, _OFF_B + i)),
        pl.BlockSpec((_BR, 128), lambda i: (_OFF_R + i, 0)),
        pl.BlockSpec((_BR, 128), lambda i: (_OFF_R + i, 0)),
        pl.BlockSpec((_BR, 128), lambda i: (_OFF_R + i, 0)),
    ],
    out_specs=pl.BlockSpec((_BR, 128), lambda i: (i, 0)),
    out_shape=jax.ShapeDtypeStruct((_R_TC // 128, 128), jnp.int32),
)


def kernel(probs, uncertainty_combined, spectral_reliability, spatial_reliability):
    pt = probs.T
    cu = uncertainty_combined.reshape(-1)
    sr = spectral_reliability.reshape(-1)
    pr = spatial_reliability.reshape(-1)
    cu2 = cu.reshape(-1, 128)
    sr2 = sr.reshape(-1, 128)
    pr2 = pr.reshape(-1, 128)
    sc_out = _sc_call(pt, cu, sr, pr)
    tc_out = _tc_call(pt, cu2, sr2, pr2)
    return jnp.concatenate([sc_out, tc_out.reshape(-1)])
